# Initial kernel scaffold; baseline (speedup 1.0000x reference)
import jax
import jax.numpy as jnp
from jax.experimental import pallas as pl


def kernel(x, Wqkv, Wout):
    raise NotImplementedError("write your pallas kernel here")


# speedup vs baseline: 1.5330x; 1.5330x over previous
import math

import jax
import jax.numpy as jnp
import numpy as np
from jax.experimental import pallas as pl
from jax.experimental.pallas import tpu as pltpu

HIDDEN_DIM = 2048
NUM_HEADS = 16
SEG = 128
ROWS = 512


def _hilbert_order(seq_len):
    grid = int(math.ceil(math.sqrt(seq_len)))
    order = []
    for row in range(grid):
        cols = range(grid) if row % 2 == 0 else range(grid - 1, -1, -1)
        for col in cols:
            lp = row * grid + col
            if lp < seq_len and len(order) < seq_len:
                order.append(lp)
    return np.array(order, dtype=np.int64)


def _fused_kernel(x_ref, wqkv_ref, wout_ref, p_ref, out_ref):
    C = HIDDEN_DIM
    H = NUM_HEADS
    hd = C // H
    scale = hd ** (-0.5)
    f32 = jnp.float32

    x = x_ref[0]
    qkv = jax.lax.dot_general(
        x, wqkv_ref[...], (((1,), (0,)), ((), ())),
        preferred_element_type=f32)
    q = qkv[:, :C].astype(jnp.bfloat16)
    k = qkv[:, C:2 * C].astype(jnp.bfloat16)
    v = qkv[:, 2 * C:].astype(jnp.bfloat16)

    p_mat = p_ref[...]

    seg_outs = []
    for s0 in range(ROWS // SEG):
        r = slice(s0 * SEG, (s0 + 1) * SEG)
        q_seg = jax.lax.dot_general(
            p_mat, q[r, :], (((1,), (0,)), ((), ())),
            preferred_element_type=f32).astype(jnp.bfloat16)
        head_outs = []
        for h in range(H):
            c = slice(h * hd, (h + 1) * hd)
            qh = q_seg[:, c]
            kh = k[r, c]
            vh = v[r, c]
            scores = jax.lax.dot_general(
                qh, kh, (((1,), (1,)), ((), ())),
                preferred_element_type=f32) * scale
            m = jnp.max(scores, axis=-1, keepdims=True)
            e = jnp.exp(scores - m)
            denom = jnp.sum(e, axis=-1, keepdims=True)
            p_attn = (e / denom).astype(jnp.bfloat16)
            head_outs.append(jax.lax.dot_general(
                p_attn, vh, (((1,), (0,)), ((), ())),
                preferred_element_type=f32))
        seg_outs.append(jnp.concatenate(head_outs, axis=1))
    attn_out = jnp.concatenate(seg_outs, axis=0).astype(jnp.bfloat16)

    out_ref[0] = jax.lax.dot_general(
        attn_out, wout_ref[...], (((1,), (0,)), ((), ())),
        preferred_element_type=f32)


def kernel(x, Wqkv, Wout):
    B, N, C = x.shape
    S = N // SEG
    assert C == HIDDEN_DIM and N % SEG == 0 and ROWS % SEG == 0

    mapping = _hilbert_order(N)
    local = mapping[:SEG]
    assert all(
        np.array_equal(mapping[s * SEG:(s + 1) * SEG] - s * SEG, local)
        for s in range(S)), "hilbert mapping is not segment-local"
    p_mat = np.zeros((SEG, SEG), dtype=np.float32)
    p_mat[np.arange(SEG), local] = 1.0

    xb = x.astype(jnp.bfloat16).reshape(B * N // ROWS, ROWS, C)
    wqkv_t = Wqkv.T.astype(jnp.bfloat16)
    wout_t = Wout.T.astype(jnp.bfloat16)
    p_b = jnp.asarray(p_mat, dtype=jnp.bfloat16)

    grid = (B * N // ROWS,)
    out = pl.pallas_call(
        _fused_kernel,
        grid=grid,
        in_specs=[
            pl.BlockSpec((1, ROWS, C), lambda i: (i, 0, 0)),
            pl.BlockSpec((C, 3 * C), lambda i: (0, 0)),
            pl.BlockSpec((C, C), lambda i: (0, 0)),
            pl.BlockSpec((SEG, SEG), lambda i: (0, 0)),
        ],
        out_specs=pl.BlockSpec((1, ROWS, C), lambda i: (i, 0, 0)),
        out_shape=jax.ShapeDtypeStruct((B * N // ROWS, ROWS, C), jnp.float32),
    )(xb, wqkv_t, wout_t, p_b)
    return out.reshape(B, N, C)


# speedup vs baseline: 2.1314x; 1.3904x over previous
import math

import jax
import jax.numpy as jnp
import numpy as np
from jax.experimental import pallas as pl
from jax.experimental.pallas import tpu as pltpu

HIDDEN_DIM = 2048
NUM_HEADS = 16
SEG = 128
ROWS = 512


def _hilbert_order(seq_len):
    grid = int(math.ceil(math.sqrt(seq_len)))
    order = []
    for row in range(grid):
        cols = range(grid) if row % 2 == 0 else range(grid - 1, -1, -1)
        for col in cols:
            lp = row * grid + col
            if lp < seq_len and len(order) < seq_len:
                order.append(lp)
    return np.array(order, dtype=np.int64)


def _fused_kernel(x_ref, wqkv_ref, wout_ref, p_ref, out_ref):
    C = HIDDEN_DIM
    H = NUM_HEADS
    hd = C // H
    f32 = jnp.float32

    x = x_ref[0]
    bf16 = jnp.bfloat16
    qkv = jax.lax.dot_general(
        x, wqkv_ref[...], (((1,), (0,)), ((), ())),
        preferred_element_type=f32).astype(bf16)
    q = qkv[:, :C]
    k = qkv[:, C:2 * C]
    v = qkv[:, 2 * C:]

    p_mat = p_ref[...]
    q_perm = [
        jax.lax.dot_general(
            p_mat, q[s0 * SEG:(s0 + 1) * SEG, :], (((1,), (0,)), ((), ())),
            preferred_element_type=f32).astype(bf16)
        for s0 in range(ROWS // SEG)
    ]

    e_list = []
    rdenom_list = []
    for s0 in range(ROWS // SEG):
        r = slice(s0 * SEG, (s0 + 1) * SEG)
        for h in range(H):
            c = slice(h * hd, (h + 1) * hd)
            scores = jax.lax.dot_general(
                q_perm[s0][:, c], k[r, c], (((1,), (1,)), ((), ())),
                preferred_element_type=f32)
            m = jnp.max(scores, axis=-1, keepdims=True)
            e = jnp.exp(scores - m)
            e_list.append(e.astype(bf16))
            rdenom_list.append(1.0 / jnp.sum(e, axis=-1, keepdims=True))

    seg_outs = []
    for s0 in range(ROWS // SEG):
        r = slice(s0 * SEG, (s0 + 1) * SEG)
        head_outs = []
        for h in range(H):
            c = slice(h * hd, (h + 1) * hd)
            idx = s0 * H + h
            o = jax.lax.dot_general(
                e_list[idx], v[r, c], (((1,), (0,)), ((), ())),
                preferred_element_type=f32)
            head_outs.append((o * rdenom_list[idx]).astype(bf16))
        seg_outs.append(jnp.concatenate(head_outs, axis=1))
    attn_out = jnp.concatenate(seg_outs, axis=0)

    out_ref[0] = jax.lax.dot_general(
        attn_out, wout_ref[...], (((1,), (0,)), ((), ())),
        preferred_element_type=f32)


def kernel(x, Wqkv, Wout):
    B, N, C = x.shape
    S = N // SEG
    assert C == HIDDEN_DIM and N % SEG == 0 and ROWS % SEG == 0

    mapping = _hilbert_order(N)
    local = mapping[:SEG]
    assert all(
        np.array_equal(mapping[s * SEG:(s + 1) * SEG] - s * SEG, local)
        for s in range(S)), "hilbert mapping is not segment-local"
    p_mat = np.zeros((SEG, SEG), dtype=np.float32)
    p_mat[np.arange(SEG), local] = 1.0

    xb = x.astype(jnp.bfloat16).reshape(B * N // ROWS, ROWS, C)
    hd = C // NUM_HEADS
    scale = hd ** (-0.5)
    wqkv_t = Wqkv.T.at[:, :C].multiply(scale).astype(jnp.bfloat16)
    wout_t = Wout.T.astype(jnp.bfloat16)
    p_b = jnp.asarray(p_mat, dtype=jnp.bfloat16)

    grid = (B * N // ROWS,)
    out = pl.pallas_call(
        _fused_kernel,
        grid=grid,
        in_specs=[
            pl.BlockSpec((1, ROWS, C), lambda i: (i, 0, 0)),
            pl.BlockSpec((C, 3 * C), lambda i: (0, 0)),
            pl.BlockSpec((C, C), lambda i: (0, 0)),
            pl.BlockSpec((SEG, SEG), lambda i: (0, 0)),
        ],
        out_specs=pl.BlockSpec((1, ROWS, C), lambda i: (i, 0, 0)),
        out_shape=jax.ShapeDtypeStruct((B * N // ROWS, ROWS, C), jnp.float32),
    )(xb, wqkv_t, wout_t, p_b)
    return out.reshape(B, N, C)


# speedup vs baseline: 2.5450x; 1.1940x over previous
import math

import jax
import jax.numpy as jnp
import numpy as np
from jax.experimental import pallas as pl
from jax.experimental.pallas import tpu as pltpu

HIDDEN_DIM = 2048
NUM_HEADS = 16
SEG = 128
ROWS = 512


def _hilbert_order(seq_len):
    grid = int(math.ceil(math.sqrt(seq_len)))
    order = []
    for row in range(grid):
        cols = range(grid) if row % 2 == 0 else range(grid - 1, -1, -1)
        for col in cols:
            lp = row * grid + col
            if lp < seq_len and len(order) < seq_len:
                order.append(lp)
    return np.array(order, dtype=np.int64)


def _fused_kernel(x_ref, wqkv_ref, wout_ref, p_ref, out_ref):
    C = HIDDEN_DIM
    H = NUM_HEADS
    hd = C // H
    f32 = jnp.float32

    bf16 = jnp.bfloat16
    x = x_ref[0].astype(bf16)
    qkv = jax.lax.dot_general(
        x, wqkv_ref[...], (((1,), (1,)), ((), ())),
        preferred_element_type=f32).astype(bf16)
    q = qkv[:, :C]
    k = qkv[:, C:2 * C]
    v = qkv[:, 2 * C:]

    p_mat = p_ref[...]
    q_perm = [
        jax.lax.dot_general(
            p_mat, q[s0 * SEG:(s0 + 1) * SEG, :], (((1,), (0,)), ((), ())),
            preferred_element_type=f32).astype(bf16)
        for s0 in range(ROWS // SEG)
    ]

    e_list = []
    rdenom_list = []
    for s0 in range(ROWS // SEG):
        r = slice(s0 * SEG, (s0 + 1) * SEG)
        for h in range(H):
            c = slice(h * hd, (h + 1) * hd)
            scores = jax.lax.dot_general(
                q_perm[s0][:, c], k[r, c], (((1,), (1,)), ((), ())),
                preferred_element_type=f32)
            m = jnp.max(scores, axis=-1, keepdims=True)
            e = jnp.exp(scores - m)
            e_list.append(e.astype(bf16))
            rdenom_list.append(1.0 / jnp.sum(e, axis=-1, keepdims=True))

    seg_outs = []
    for s0 in range(ROWS // SEG):
        r = slice(s0 * SEG, (s0 + 1) * SEG)
        head_outs = []
        for h in range(H):
            c = slice(h * hd, (h + 1) * hd)
            idx = s0 * H + h
            o = jax.lax.dot_general(
                e_list[idx], v[r, c], (((1,), (0,)), ((), ())),
                preferred_element_type=f32)
            head_outs.append((o * rdenom_list[idx]).astype(bf16))
        seg_outs.append(jnp.concatenate(head_outs, axis=1))
    attn_out = jnp.concatenate(seg_outs, axis=0)

    out_ref[0] = jax.lax.dot_general(
        attn_out, wout_ref[...], (((1,), (1,)), ((), ())),
        preferred_element_type=f32)


def kernel(x, Wqkv, Wout):
    B, N, C = x.shape
    S = N // SEG
    assert C == HIDDEN_DIM and N % SEG == 0 and ROWS % SEG == 0

    mapping = _hilbert_order(N)
    local = mapping[:SEG]
    assert all(
        np.array_equal(mapping[s * SEG:(s + 1) * SEG] - s * SEG, local)
        for s in range(S)), "hilbert mapping is not segment-local"
    p_mat = np.zeros((SEG, SEG), dtype=np.float32)
    p_mat[np.arange(SEG), local] = 1.0

    xb = x.reshape(B * N // ROWS, ROWS, C)
    hd = C // NUM_HEADS
    scale = hd ** (-0.5)
    wqkv_b = Wqkv.at[:C, :].multiply(scale).astype(jnp.bfloat16)
    wout_b = Wout.astype(jnp.bfloat16)
    p_b = jnp.asarray(p_mat, dtype=jnp.bfloat16)

    grid = (B * N // ROWS,)
    out = pl.pallas_call(
        _fused_kernel,
        grid=grid,
        in_specs=[
            pl.BlockSpec((1, ROWS, C), lambda i: (i, 0, 0)),
            pl.BlockSpec((3 * C, C), lambda i: (0, 0)),
            pl.BlockSpec((C, C), lambda i: (0, 0)),
            pl.BlockSpec((SEG, SEG), lambda i: (0, 0)),
        ],
        out_specs=pl.BlockSpec((1, ROWS, C), lambda i: (i, 0, 0)),
        out_shape=jax.ShapeDtypeStruct((B * N // ROWS, ROWS, C), jnp.float32),
    )(xb, wqkv_b, wout_b, p_b)
    return out.reshape(B, N, C)


# speedup vs baseline: 2.8535x; 1.1212x over previous
import math

import jax
import jax.numpy as jnp
import numpy as np
from jax.experimental import pallas as pl
from jax.experimental.pallas import tpu as pltpu

HIDDEN_DIM = 2048
NUM_HEADS = 16
SEG = 128
ROWS = 512


def _hilbert_order(seq_len):
    grid = int(math.ceil(math.sqrt(seq_len)))
    order = []
    for row in range(grid):
        cols = range(grid) if row % 2 == 0 else range(grid - 1, -1, -1)
        for col in cols:
            lp = row * grid + col
            if lp < seq_len and len(order) < seq_len:
                order.append(lp)
    return np.array(order, dtype=np.int64)


def _fused_kernel(x_ref, wqkv_ref, wout_ref, p_ref, out_ref):
    C = HIDDEN_DIM
    H = NUM_HEADS
    hd = C // H
    f32 = jnp.float32

    bf16 = jnp.bfloat16
    x = x_ref[0].astype(bf16)
    qkv = jax.lax.dot_general(
        x, wqkv_ref[...], (((1,), (1,)), ((), ())),
        preferred_element_type=f32).astype(bf16)
    q = qkv[:, :C]
    k = qkv[:, C:2 * C]
    v = qkv[:, 2 * C:]

    p_mat = p_ref[...]
    q_perm = [
        jax.lax.dot_general(
            p_mat, q[s0 * SEG:(s0 + 1) * SEG, :], (((1,), (0,)), ((), ())),
            preferred_element_type=f32).astype(bf16)
        for s0 in range(ROWS // SEG)
    ]

    e_list = []
    rdenom_list = []
    for s0 in range(ROWS // SEG):
        r = slice(s0 * SEG, (s0 + 1) * SEG)
        for h in range(H):
            c = slice(h * hd, (h + 1) * hd)
            scores = jax.lax.dot_general(
                q_perm[s0][:, c], k[r, c], (((1,), (1,)), ((), ())),
                preferred_element_type=f32)
            m = jnp.max(scores, axis=-1, keepdims=True)
            e = jnp.exp(scores - m)
            e_list.append(e.astype(bf16))
            rdenom_list.append(1.0 / jnp.sum(e, axis=-1, keepdims=True))

    seg_outs = []
    for s0 in range(ROWS // SEG):
        r = slice(s0 * SEG, (s0 + 1) * SEG)
        head_outs = []
        for h in range(H):
            c = slice(h * hd, (h + 1) * hd)
            idx = s0 * H + h
            o = jax.lax.dot_general(
                e_list[idx], v[r, c], (((1,), (0,)), ((), ())),
                preferred_element_type=f32)
            head_outs.append((o * rdenom_list[idx]).astype(bf16))
        seg_outs.append(jnp.concatenate(head_outs, axis=1))
    attn_out = jnp.concatenate(seg_outs, axis=0)

    out_ref[0] = jax.lax.dot_general(
        attn_out, wout_ref[...], (((1,), (1,)), ((), ())),
        preferred_element_type=f32)


def kernel(x, Wqkv, Wout):
    B, N, C = x.shape
    S = N // SEG
    assert C == HIDDEN_DIM and N % SEG == 0 and ROWS % SEG == 0

    mapping = _hilbert_order(N)
    local = mapping[:SEG]
    assert all(
        np.array_equal(mapping[s * SEG:(s + 1) * SEG] - s * SEG, local)
        for s in range(S)), "hilbert mapping is not segment-local"
    p_mat = np.zeros((SEG, SEG), dtype=np.float32)
    p_mat[np.arange(SEG), local] = 1.0

    xb = x.reshape(B * N // ROWS, ROWS, C)
    hd = C // NUM_HEADS
    scale = hd ** (-0.5)
    wqkv_b = Wqkv.astype(jnp.bfloat16)
    wout_b = Wout.astype(jnp.bfloat16)
    p_b = jnp.asarray(p_mat * scale, dtype=jnp.bfloat16)

    grid = (B * N // ROWS,)
    out = pl.pallas_call(
        _fused_kernel,
        grid=grid,
        in_specs=[
            pl.BlockSpec((1, ROWS, C), lambda i: (i, 0, 0)),
            pl.BlockSpec((3 * C, C), lambda i: (0, 0)),
            pl.BlockSpec((C, C), lambda i: (0, 0)),
            pl.BlockSpec((SEG, SEG), lambda i: (0, 0)),
        ],
        out_specs=pl.BlockSpec((1, ROWS, C), lambda i: (i, 0, 0)),
        out_shape=jax.ShapeDtypeStruct((B * N // ROWS, ROWS, C), jnp.float32),
        compiler_params=pltpu.CompilerParams(
            dimension_semantics=("parallel",)),
    )(xb, wqkv_b, wout_b, p_b)
    return out.reshape(B, N, C)
